# baseline (speedup 1.0000x reference)
import functools

import jax
import jax.numpy as jnp
from jax import lax
from jax.experimental import pallas as pl
from jax.experimental.pallas import tpu as pltpu
from jax.experimental.pallas import tpu_sc as plsc

_VOCAB = 1000
_BATCH = 16384
_NC = 2
_NS = 16
_NW = _NC * _NS
_BPW = _BATCH // _NW
_CHUNK = 64
_NCHUNK = _BPW // _CHUNK

_mesh = plsc.VectorSubcoreMesh(core_axis_name="c", subcore_axis_name="s")


@functools.partial(
    pl.kernel,
    out_type=jax.ShapeDtypeStruct((_BATCH, _VOCAB), jnp.float32),
    mesh=_mesh,
    scratch_types=[
        pltpu.VMEM((_BPW,), jnp.int32),
        pltpu.VMEM((_CHUNK, _VOCAB), jnp.float32),
        pltpu.VMEM((_CHUNK, _VOCAB), jnp.float32),
        pltpu.SemaphoreType.DMA,
        pltpu.SemaphoreType.DMA,
    ],
    compiler_params=pltpu.CompilerParams(use_tc_tiling_on_sc=False),
)
def _gather_kernel(table_hbm, idx_hbm, out_hbm, idx_v, buf0, buf1, sem0, sem1):
    wid = lax.axis_index("s") * _NC + lax.axis_index("c")
    base = wid * _BPW
    pltpu.sync_copy(idx_hbm.at[pl.ds(base, _BPW)], idx_v)
    bufs = (buf0, buf1)
    sems = (sem0, sem1)
    copies = [
        pltpu.async_copy(table_hbm.at[idx_v.at[pl.ds(0, _CHUNK)]], buf0, sem0),
        None,
    ]
    for j in range(_NCHUNK):
        cur, nxt = j % 2, (j + 1) % 2
        if j + 1 < _NCHUNK:
            copies[nxt] = pltpu.async_copy(
                table_hbm.at[idx_v.at[pl.ds((j + 1) * _CHUNK, _CHUNK)]],
                bufs[nxt],
                sems[nxt],
            )
        copies[cur].wait()
        pltpu.sync_copy(bufs[cur], out_hbm.at[pl.ds(base + j * _CHUNK, _CHUNK)])


def kernel(idx, embedding):
    return _gather_kernel(embedding, idx)
